# baseline (speedup 1.0000x reference)
import jax
import jax.numpy as jnp
from jax.experimental import pallas as pl
from jax.experimental.pallas import tpu as pltpu

_B = 16384
_K = 4
_G = 8
_CH = _B // (_K * _G)


def _mlp_kernel(x0, x1, x2, x3, w1_ref, b1_ref, w2_ref, b2_ref, o_ref):
    w1 = w1_ref[...]
    b1v = b1_ref[...]
    w2 = w2_ref[...]
    b2v = b2_ref[0, 0]
    for k, xk in enumerate((x0, x1, x2, x3)):
        h = jnp.dot(xk[...], w1, preferred_element_type=jnp.float32)
        h = jnp.maximum(h + b1v, 0.0)
        logit = jnp.dot(h, w2, preferred_element_type=jnp.float32)
        o_ref[pl.ds(k * _CH, _CH), :] = 0.5 * jnp.tanh(0.5 * (logit + b2v)) + 0.5


@jax.jit
def kernel(x, W1, b1, W2, b2):
    B, D = x.shape
    H = W1.shape[1]
    b1r = b1.reshape(1, H)
    b2r = b2.reshape(1, 1)
    out = pl.pallas_call(
        _mlp_kernel,
        grid=(_G,),
        in_specs=[
            pl.BlockSpec((_CH, D), lambda i, k=k: (_K * i + k, 0))
            for k in range(_K)
        ]
        + [
            pl.BlockSpec((D, H), lambda i: (0, 0)),
            pl.BlockSpec((1, H), lambda i: (0, 0)),
            pl.BlockSpec((D, 1), lambda i: (0, 0)),
            pl.BlockSpec((1, 1), lambda i: (0, 0)),
        ],
        out_specs=pl.BlockSpec((_K * _CH, 1), lambda i: (i, 0)),
        out_shape=jax.ShapeDtypeStruct((B, 1), jnp.float32),
        compiler_params=pltpu.CompilerParams(
            dimension_semantics=("parallel",)
        ),
    )(x, x, x, x, W1, b1r, W2, b2r)
    return out


# speedup vs baseline: 2.1810x; 2.1810x over previous
import jax
import jax.numpy as jnp
from jax.experimental import pallas as pl
from jax.experimental.pallas import tpu as pltpu

_B = 16384
_K = 4
_G = 4
_CH = _B // (_K * _G)


def _mlp_kernel(x0, x1, x2, x3, w1_ref, b1_ref, w2_ref, b2_ref, o_ref):
    w1 = w1_ref[...]
    b1v = b1_ref[...]
    w2 = w2_ref[...]
    b2v = b2_ref[...]
    for k, xk in enumerate((x0, x1, x2, x3)):
        h = jnp.dot(xk[...], w1, preferred_element_type=jnp.float32)
        h = jnp.maximum(h + b1v, 0.0)
        logit = jax.lax.dot_general(
            w2, h, (((1,), (1,)), ((), ())), preferred_element_type=jnp.float32
        )
        o_ref[0, pl.ds(k, 1), :] = 0.5 * jnp.tanh(0.5 * (logit + b2v)) + 0.5


@jax.jit
def kernel(x, W1, b1, W2, b2):
    B, D = x.shape
    H = W1.shape[1]
    b1r = b1.reshape(1, H)
    w2r = W2.reshape(1, H)
    b2r = b2.reshape(1, 1)
    out = pl.pallas_call(
        _mlp_kernel,
        grid=(_G,),
        in_specs=[
            pl.BlockSpec((_CH, D), lambda i, k=k: (_K * i + k, 0))
            for k in range(_K)
        ]
        + [
            pl.BlockSpec((D, H), lambda i: (0, 0)),
            pl.BlockSpec((1, H), lambda i: (0, 0)),
            pl.BlockSpec((1, H), lambda i: (0, 0)),
            pl.BlockSpec((1, 1), lambda i: (0, 0)),
        ],
        out_specs=pl.BlockSpec((1, _K, _CH), lambda i: (i, 0, 0)),
        out_shape=jax.ShapeDtypeStruct((_G, _K, _CH), jnp.float32),
        compiler_params=pltpu.CompilerParams(
            dimension_semantics=("parallel",)
        ),
    )(x, x, x, x, W1, b1r, w2r, b2r)
    return out.reshape(B, 1)


# speedup vs baseline: 2.3897x; 1.0957x over previous
import jax
import jax.numpy as jnp
from jax.experimental import pallas as pl
from jax.experimental.pallas import tpu as pltpu

_B = 16384
_K = 4
_G = 2
_CH = _B // (_K * _G)


def _mlp_kernel(x0, x1, x2, x3, w1_ref, b1_ref, w2_ref, b2_ref, o_ref):
    w1 = w1_ref[...]
    b1v = b1_ref[...]
    w2 = w2_ref[...]
    b2v = b2_ref[...]
    for k, xk in enumerate((x0, x1, x2, x3)):
        h = jnp.dot(xk[...], w1, preferred_element_type=jnp.float32)
        h = jnp.maximum(h + b1v, 0.0)
        logit = jax.lax.dot_general(
            w2, h, (((1,), (1,)), ((), ())), preferred_element_type=jnp.float32
        )
        o_ref[0, pl.ds(k, 1), :] = 0.5 * jnp.tanh(0.5 * (logit + b2v)) + 0.5


@jax.jit
def kernel(x, W1, b1, W2, b2):
    B, D = x.shape
    H = W1.shape[1]
    b1r = b1.reshape(1, H)
    w2r = W2.reshape(1, H)
    b2r = b2.reshape(1, 1)
    out = pl.pallas_call(
        _mlp_kernel,
        grid=(_G,),
        in_specs=[
            pl.BlockSpec((_CH, D), lambda i, k=k: (_K * i + k, 0))
            for k in range(_K)
        ]
        + [
            pl.BlockSpec((D, H), lambda i: (0, 0)),
            pl.BlockSpec((1, H), lambda i: (0, 0)),
            pl.BlockSpec((1, H), lambda i: (0, 0)),
            pl.BlockSpec((1, 1), lambda i: (0, 0)),
        ],
        out_specs=pl.BlockSpec((1, _K, _CH), lambda i: (i, 0, 0)),
        out_shape=jax.ShapeDtypeStruct((_G, _K, _CH), jnp.float32),
        compiler_params=pltpu.CompilerParams(
            dimension_semantics=("parallel",)
        ),
    )(x, x, x, x, W1, b1r, w2r, b2r)
    return out.reshape(B, 1)
